# baseline (speedup 1.0000x reference)
import functools

import jax
import jax.numpy as jnp
from jax import lax
from jax.experimental import pallas as pl
from jax.experimental.pallas import tpu as pltpu
from jax.experimental.pallas import tpu_sc as plsc

_FEAT_DIM = 64
_NUM_CLASSES = 100000
_BATCH = 16384
_LAMBDA_C = 1.0

_NC = 2
_NS = 16
_L = 16
_NW = _NC * _NS
_B_PER_W = _BATCH // _NW
_IDX_CHUNK = 128
_N_CHUNKS = _B_PER_W // _IDX_CHUNK


def _sc_partials(feat, label, centers):
    mesh = plsc.VectorSubcoreMesh(core_axis_name="c", subcore_axis_name="s")

    @functools.partial(
        pl.kernel,
        mesh=mesh,
        out_type=jax.ShapeDtypeStruct((_NW, _L), jnp.float32),
        scratch_types=[
            pltpu.VMEM((_N_CHUNKS, _IDX_CHUNK), jnp.int32),
            pltpu.VMEM((_B_PER_W, _FEAT_DIM), jnp.float32),
            pltpu.VMEM((_B_PER_W, _FEAT_DIM), jnp.float32),
            pltpu.VMEM((_L,), jnp.float32),
            pltpu.SemaphoreType.DMA,
        ],
        compiler_params=pltpu.CompilerParams(use_tc_tiling_on_sc=False),
    )
    def k(feat_hbm, label_hbm, centers_hbm, out_hbm,
          idx_v, feat_v, cent_v, acc_v, sem):
        wid = lax.axis_index("s") * _NC + lax.axis_index("c")
        base = wid * _B_PER_W
        pltpu.sync_copy(label_hbm.at[wid], idx_v)
        copies = []
        for c in range(_N_CHUNKS):
            copies.append(pltpu.async_copy(
                centers_hbm.at[idx_v.at[c]],
                cent_v.at[pl.ds(c * _IDX_CHUNK, _IDX_CHUNK)],
                sem,
            ))
        pltpu.sync_copy(feat_hbm.at[pl.ds(base, _B_PER_W)], feat_v)
        for cp in copies:
            cp.wait()

        def row(i, acc):
            for j in range(_FEAT_DIM // _L):
                f = feat_v[i, pl.ds(j * _L, _L)]
                c = cent_v[i, pl.ds(j * _L, _L)]
                d = f - c
                acc = acc + d * d
            return acc

        acc = lax.fori_loop(0, _B_PER_W, row, jnp.zeros((_L,), jnp.float32))
        acc_v[...] = acc
        pltpu.sync_copy(acc_v, out_hbm.at[wid])

    return k(feat, label, centers)


def _finish_body(p_ref, o_ref):
    s = jnp.sum(p_ref[...])
    o_ref[0, 0] = _LAMBDA_C / 2.0 / _BATCH * jnp.sqrt(s)


def kernel(feat, label, centers):
    label_r = label.astype(jnp.int32).reshape(_NW, _N_CHUNKS, _IDX_CHUNK)
    partials = _sc_partials(feat, label_r, centers)
    loss = pl.pallas_call(
        _finish_body,
        out_shape=jax.ShapeDtypeStruct((1, 1), jnp.float32),
        out_specs=pl.BlockSpec(memory_space=pltpu.SMEM),
    )(partials)
    return loss[0, 0]


# speedup vs baseline: 1.3135x; 1.3135x over previous
import functools

import jax
import jax.numpy as jnp
from jax import lax
from jax.experimental import pallas as pl
from jax.experimental.pallas import tpu as pltpu
from jax.experimental.pallas import tpu_sc as plsc

_FEAT_DIM = 64
_NUM_CLASSES = 100000
_BATCH = 16384
_LAMBDA_C = 1.0

_NC = 2
_NS = 16
_L = 16
_NW = _NC * _NS
_B_PER_W = _BATCH // _NW
_N_PASS = 2
_B_PASS = _B_PER_W // _N_PASS


def _sc_partials(feat, label, centers):
    mesh = plsc.VectorSubcoreMesh(core_axis_name="c", subcore_axis_name="s")

    @functools.partial(
        pl.kernel,
        mesh=mesh,
        out_type=jax.ShapeDtypeStruct((_NW, _L), jnp.float32),
        scratch_types=[
            pltpu.VMEM((_B_PER_W,), jnp.int32),
            pltpu.VMEM((_B_PASS, _FEAT_DIM), jnp.float32),
            pltpu.VMEM((_B_PASS, _FEAT_DIM), jnp.float32),
            pltpu.VMEM((_L,), jnp.float32),
            pltpu.SemaphoreType.DMA,
        ],
        compiler_params=pltpu.CompilerParams(use_tc_tiling_on_sc=True),
    )
    def k(feat_hbm, label_hbm, centers_hbm, out_hbm,
          idx_v, feat_v, cent_v, acc_v, sem):
        wid = lax.axis_index("s") * _NC + lax.axis_index("c")
        base = wid * _B_PER_W
        pltpu.sync_copy(label_hbm.at[wid], idx_v)

        acc = jnp.zeros((_L,), jnp.float32)
        for p in range(_N_PASS):
            def fire(g, _, p=p):
                lv = idx_v[pl.ds(p * _B_PASS + g * _L, _L)]
                for u in range(_L):
                    pltpu.async_copy(
                        centers_hbm.at[pl.ds(lv[u], 1)],
                        cent_v.at[pl.ds(g * _L + u, 1)],
                        sem,
                    )
                return _

            lax.fori_loop(0, _B_PASS // _L, fire, 0)

            pltpu.sync_copy(
                feat_hbm.at[pl.ds(base + p * _B_PASS, _B_PASS)], feat_v)

            pltpu.make_async_copy(
                feat_hbm.at[pl.ds(0, _B_PASS)], cent_v, sem,
            ).wait()

            def row(i, acc):
                for j in range(_FEAT_DIM // _L):
                    f = feat_v[i, pl.ds(j * _L, _L)]
                    c = cent_v[i, pl.ds(j * _L, _L)]
                    d = f - c
                    acc = acc + d * d
                return acc

            acc = lax.fori_loop(0, _B_PASS, row, acc)
        acc_v[...] = acc
        pltpu.sync_copy(acc_v, out_hbm.at[wid])

    return k(feat, label, centers)


def _finish_body(p_ref, o_ref):
    s = jnp.sum(p_ref[...])
    o_ref[0, 0] = _LAMBDA_C / 2.0 / _BATCH * jnp.sqrt(s)


def kernel(feat, label, centers):
    label_r = label.astype(jnp.int32).reshape(_NW, _B_PER_W)
    partials = _sc_partials(feat, label_r, centers)
    loss = pl.pallas_call(
        _finish_body,
        out_shape=jax.ShapeDtypeStruct((1, 1), jnp.float32),
        out_specs=pl.BlockSpec(memory_space=pltpu.SMEM),
    )(partials)
    return loss[0, 0]
